# baseline (speedup 1.0000x reference)
import jax
import jax.numpy as jnp
from jax import lax
from jax.experimental import pallas as pl
from jax.experimental.pallas import tpu as pltpu
from jax.experimental.pallas import tpu_sc as plsc

E = 8
D = 1024
F = 2048
T = 2048
BT = 256
K = 3
NT = (K * T + E * (BT - 1) + BT - 1) // BT
S_MAX = NT * BT
NC = 2
NS = 16
NW = NC * NS
CT = T // NW
CH = 16
NCH = CT // CH


def _cumsum_lanes(a, width):
    sh = 1
    while sh < width:
        z = jnp.zeros(a.shape[:1] + (sh,), a.dtype)
        a = a + jnp.concatenate([z, a[:, :-sh]], axis=1)
        sh *= 2
    return a


def _cumsum_rows(a):
    n = a.shape[0]
    sh = 1
    while sh < n:
        z = jnp.zeros((sh,) + a.shape[1:], a.dtype)
        a = a + jnp.concatenate([z, a[:-sh]], axis=0)
        sh *= 2
    return a


def _route_body(x_ref, wg_ref, wrows_ref, scat_ref, gath_ref, meta_ref):
    logits = jnp.dot(x_ref[...], wg_ref[...], preferred_element_type=jnp.float32)
    lane = jax.lax.broadcasted_iota(jnp.int32, logits.shape, 1)
    valid = lane < E
    logits = jnp.where(valid, logits, jnp.float32(-1e30))
    lmax = jnp.max(logits, axis=1, keepdims=True)
    p = jnp.exp(logits - lmax)
    p = jnp.where(valid, p, 0.0)
    probs = p / jnp.sum(p, axis=1, keepdims=True)
    w0 = jnp.max(probs, axis=1, keepdims=True)
    cond0 = (probs == w0) & valid
    probs_m = jnp.where(cond0, 0.0, probs)
    w1v = jnp.max(probs_m, axis=1, keepdims=True)
    cond1 = (probs_m == w1v) & valid
    m = 1.0 / (w0 + w1v)
    weights = m * w0 * cond0.astype(jnp.float32) + m * w1v * cond1.astype(jnp.float32)

    sel = (cond0 | cond1).astype(jnp.int32)
    eexcl = _cumsum_lanes(sel, E) - sel
    sel3 = sel * (eexcl < K).astype(jnp.int32)
    tincl = _cumsum_rows(sel3)
    rk = tincl - sel3
    counts = tincl[-1:, :]
    pc = (counts + (BT - 1)) // BT * BT
    goff = _cumsum_lanes(pc, E) - pc
    slot = goff + rk

    tok = jax.lax.broadcasted_iota(jnp.int32, (T, 1), 0)
    scat = jnp.zeros((T, 128), jnp.int32)
    gath = jnp.zeros((T, 128), jnp.int32)
    for k in range(K):
        hot = (sel3 * (eexcl == k).astype(jnp.int32)).astype(jnp.float32)
        live = jnp.sum(hot, axis=1, keepdims=True) > 0
        slot_k = jnp.sum(hot * slot.astype(jnp.float32), axis=1,
                         keepdims=True).astype(jnp.int32)
        w_k = jnp.sum(hot * weights, axis=1, keepdims=True)
        w_k = jnp.where(live, w_k, 0.0)
        scat_k = jnp.where(live, slot_k, S_MAX + BT + tok)
        gath_k = jnp.where(live, slot_k, S_MAX)
        wrows_ref[k] = jnp.broadcast_to(w_k, (T, 128))
        is_k = lane == k
        scat = jnp.where(is_k, scat_k, scat)
        gath = jnp.where(is_k, gath_k, gath)
    scat_ref[...] = scat
    gath_ref[...] = gath

    jv = jax.lax.broadcasted_iota(jnp.int32, (NT + 1, 128), 0) * BT
    mlane = jax.lax.broadcasted_iota(jnp.int32, (NT + 1, 128), 1)
    ind = ((goff <= jv) & (jv < goff + pc) & (mlane < E)).astype(jnp.int32)
    te_row = jnp.sum(ind * mlane, axis=1, keepdims=True)
    active = jnp.sum(ind, axis=1, keepdims=True)
    n_act = jnp.sum(active, axis=0, keepdims=True)
    jrow = jax.lax.broadcasted_iota(jnp.int32, (NT + 1, 1), 0)
    last = n_act - 1
    rowblk = jnp.where(active > 0, jrow, last)
    te_last = jnp.sum(te_row * (jrow == last).astype(jnp.int32), axis=0,
                      keepdims=True)
    te_c = jnp.where(active > 0, te_row, te_last)
    rowblk = jnp.where(jrow == NT, NT, rowblk)
    mode = jnp.where(jrow == NT, 2, active)
    meta = jnp.where(mlane == 0, te_c, 0)
    meta = jnp.where(mlane == 1, rowblk, meta)
    meta = jnp.where(mlane == 2, mode, meta)
    meta_ref[...] = meta


def _dispatch_body(x_hbm, scat_hbm, wrows_hbm, xg_hbm, ws_hbm,
                   idx_v, rows_v, wr_v, sem):
    wid = lax.axis_index("s") * NC + lax.axis_index("c")
    base = wid * CT
    pltpu.sync_copy(scat_hbm.at[wid], idx_v)
    stage = [pltpu.async_copy(x_hbm.at[pl.ds(base, CT)], rows_v, sem)]
    stage += [pltpu.async_copy(wrows_hbm.at[k, pl.ds(base, CT)], wr_v.at[k], sem)
              for k in range(K)]
    for c in stage:
        c.wait()
    copies = [pltpu.async_copy(rows_v, xg_hbm.at[idx_v.at[k]], sem)
              for k in range(K)]
    copies += [pltpu.async_copy(wr_v.at[k], ws_hbm.at[idx_v.at[k]], sem)
               for k in range(K)]
    for c in copies:
        c.wait()


def _sc_dispatch(x, scat3, wrows):
    mesh = plsc.VectorSubcoreMesh(core_axis_name="c", subcore_axis_name="s")
    return pl.kernel(
        _dispatch_body,
        out_type=[
            jax.ShapeDtypeStruct((S_MAX + BT + T, D), jnp.float32),
            jax.ShapeDtypeStruct((S_MAX + BT + T, 128), jnp.float32),
        ],
        mesh=mesh,
        scratch_types=[
            pltpu.VMEM((K, CT), jnp.int32),
            pltpu.VMEM((CT, D), jnp.float32),
            pltpu.VMEM((K, CT, 128), jnp.float32),
            pltpu.SemaphoreType.DMA,
        ],
    )(x, scat3, wrows)


def _ffn_body(te_ref, rb_ref, tv_ref, xg_ref, ws_ref, w1_ref, w3_ref, w2_ref,
              out_ref):
    j = pl.program_id(0)

    @pl.when(tv_ref[j] == 1)
    def _():
        xb = xg_ref[...].astype(jnp.bfloat16)
        h = jnp.dot(xb, w1_ref[0], preferred_element_type=jnp.float32)
        g = jnp.dot(xb, w3_ref[0], preferred_element_type=jnp.float32)
        act = (h * jax.nn.sigmoid(h)) * g
        out_ref[...] = jnp.dot(act.astype(jnp.bfloat16), w2_ref[0],
                               preferred_element_type=jnp.float32) * ws_ref[:, 0:1]

    @pl.when(tv_ref[j] == 2)
    def _():
        out_ref[...] = jnp.zeros((BT, D), jnp.float32)


def _grouped_ffn(xg, ws, w1b, w3b, w2b, te, rb, tv):
    grid_spec = pltpu.PrefetchScalarGridSpec(
        num_scalar_prefetch=3,
        grid=(NT + 1,),
        in_specs=[
            pl.BlockSpec((BT, D), lambda j, te, rb, tv: (rb[j], 0)),
            pl.BlockSpec((BT, 128), lambda j, te, rb, tv: (rb[j], 0)),
            pl.BlockSpec((1, D, F), lambda j, te, rb, tv: (te[j], 0, 0)),
            pl.BlockSpec((1, D, F), lambda j, te, rb, tv: (te[j], 0, 0)),
            pl.BlockSpec((1, F, D), lambda j, te, rb, tv: (te[j], 0, 0)),
        ],
        out_specs=pl.BlockSpec((BT, D), lambda j, te, rb, tv: (rb[j], 0)),
    )
    return pl.pallas_call(
        _ffn_body,
        grid_spec=grid_spec,
        out_shape=jax.ShapeDtypeStruct((S_MAX + BT, D), jnp.float32),
    )(te, rb, tv, xg, ws, w1b, w3b, w2b)


def _combine_body(ffn_hbm, gath_hbm, out_hbm, idx_v, rows_v, out_v, sem):
    wid = lax.axis_index("s") * NC + lax.axis_index("c")
    base = wid * CT
    pltpu.sync_copy(gath_hbm.at[wid], idx_v)

    def fire(c, buf):
        return [pltpu.async_copy(
            ffn_hbm.at[idx_v.at[k, pl.ds(c * CH, CH)]],
            rows_v.at[buf, k], sem) for k in range(K)]

    pending = fire(0, 0)
    for c in range(NCH):
        nxt = fire(c + 1, (c + 1) % 2) if c + 1 < NCH else []
        for cp in pending:
            cp.wait()
        pending = nxt
        buf = c % 2
        for i in range(CH):
            def gbody(gq, _):
                for u in range(4):
                    s = pl.ds((gq * 4 + u) * 16, 16)
                    out_v[i, s] = (rows_v[buf, 0, i, s] + rows_v[buf, 1, i, s]
                                   + rows_v[buf, 2, i, s])
                return 0
            lax.fori_loop(0, D // 64, gbody, 0)
        pltpu.sync_copy(out_v, out_hbm.at[pl.ds(base + c * CH, CH)])


def _sc_combine(ffn_out, gath3):
    mesh = plsc.VectorSubcoreMesh(core_axis_name="c", subcore_axis_name="s")
    return pl.kernel(
        _combine_body,
        out_type=jax.ShapeDtypeStruct((T, D), jnp.float32),
        mesh=mesh,
        scratch_types=[
            pltpu.VMEM((K, CT), jnp.int32),
            pltpu.VMEM((2, K, CH, D), jnp.float32),
            pltpu.VMEM((CH, D), jnp.float32),
            pltpu.SemaphoreType.DMA,
        ],
    )(ffn_out, gath3)


def kernel(x, Wg, W1, W3, W2):
    wg_pad = jnp.pad(Wg, ((0, 0), (0, 128 - E)))
    wrows, scat, gath, meta = pl.pallas_call(
        _route_body,
        out_shape=[
            jax.ShapeDtypeStruct((K, T, 128), jnp.float32),
            jax.ShapeDtypeStruct((T, 128), jnp.int32),
            jax.ShapeDtypeStruct((T, 128), jnp.int32),
            jax.ShapeDtypeStruct((NT + 1, 128), jnp.int32),
        ],
    )(x, wg_pad)

    scat3 = scat[:, :K].T.reshape(K, NW, CT).transpose(1, 0, 2)
    gath3 = gath[:, :K].T.reshape(K, NW, CT).transpose(1, 0, 2)
    te = meta[:, 0]
    rb = meta[:, 1]
    tv = meta[:, 2]

    xg, ws = _sc_dispatch(x, scat3, wrows)
    w1b = W1.astype(jnp.bfloat16)
    w3b = W3.astype(jnp.bfloat16)
    w2b = W2.astype(jnp.bfloat16)
    ffn_out = _grouped_ffn(xg, ws, w1b, w3b, w2b, te, rb, tv)
    return _sc_combine(ffn_out, gath3)


# speedup vs baseline: 1.2591x; 1.2591x over previous
import jax
import jax.numpy as jnp
from jax import lax
from jax.experimental import pallas as pl
from jax.experimental.pallas import tpu as pltpu
from jax.experimental.pallas import tpu_sc as plsc

E = 8
D = 1024
F = 2048
T = 2048
BT = 256
K = 3
NT = (K * T + E * (BT - 1) + BT - 1) // BT
S_MAX = NT * BT
NC = 2
NS = 16
NW = NC * NS
CT = T // NW
CH = 16
NCH = CT // CH


def _cumsum_lanes(a, width):
    sh = 1
    while sh < width:
        z = jnp.zeros(a.shape[:1] + (sh,), a.dtype)
        a = a + jnp.concatenate([z, a[:, :-sh]], axis=1)
        sh *= 2
    return a


def _cumsum_rows(a):
    n = a.shape[0]
    sh = 1
    while sh < n:
        z = jnp.zeros((sh,) + a.shape[1:], a.dtype)
        a = a + jnp.concatenate([z, a[:-sh]], axis=0)
        sh *= 2
    return a


def _route_body(x_ref, wg_ref, wrows_ref, scat_ref, gath_ref, meta_ref):
    logits = jnp.dot(x_ref[...], wg_ref[...], preferred_element_type=jnp.float32)
    lane = jax.lax.broadcasted_iota(jnp.int32, logits.shape, 1)
    valid = lane < E
    logits = jnp.where(valid, logits, jnp.float32(-1e30))
    lmax = jnp.max(logits, axis=1, keepdims=True)
    p = jnp.exp(logits - lmax)
    p = jnp.where(valid, p, 0.0)
    probs = p / jnp.sum(p, axis=1, keepdims=True)
    w0 = jnp.max(probs, axis=1, keepdims=True)
    cond0 = (probs == w0) & valid
    probs_m = jnp.where(cond0, 0.0, probs)
    w1v = jnp.max(probs_m, axis=1, keepdims=True)
    cond1 = (probs_m == w1v) & valid
    m = 1.0 / (w0 + w1v)
    weights = m * w0 * cond0.astype(jnp.float32) + m * w1v * cond1.astype(jnp.float32)

    sel = (cond0 | cond1).astype(jnp.int32)
    eexcl = _cumsum_lanes(sel, E) - sel
    sel3 = sel * (eexcl < K).astype(jnp.int32)
    tincl = _cumsum_rows(sel3)
    rk = tincl - sel3
    counts = tincl[-1:, :]
    pc = (counts + (BT - 1)) // BT * BT
    goff = _cumsum_lanes(pc, E) - pc
    slot = goff + rk

    tok = jax.lax.broadcasted_iota(jnp.int32, (T, 1), 0)
    scat = jnp.zeros((T, 128), jnp.int32)
    gath = jnp.zeros((T, 128), jnp.int32)
    for k in range(K):
        hot = (sel3 * (eexcl == k).astype(jnp.int32)).astype(jnp.float32)
        live = jnp.sum(hot, axis=1, keepdims=True) > 0
        slot_k = jnp.sum(hot * slot.astype(jnp.float32), axis=1,
                         keepdims=True).astype(jnp.int32)
        w_k = jnp.sum(hot * weights, axis=1, keepdims=True)
        w_k = jnp.where(live, w_k, 0.0)
        scat_k = jnp.where(live, slot_k, S_MAX + BT + tok)
        gath_k = jnp.where(live, slot_k, S_MAX + (tok % BT))
        wrows_ref[k] = jnp.broadcast_to(w_k, (T, 128))
        is_k = lane == k
        scat = jnp.where(is_k, scat_k, scat)
        gath = jnp.where(is_k, gath_k, gath)
    scat_ref[...] = scat
    gath_ref[...] = gath

    jv = jax.lax.broadcasted_iota(jnp.int32, (NT + 1, 128), 0) * BT
    mlane = jax.lax.broadcasted_iota(jnp.int32, (NT + 1, 128), 1)
    ind = ((goff <= jv) & (jv < goff + pc) & (mlane < E)).astype(jnp.int32)
    te_row = jnp.sum(ind * mlane, axis=1, keepdims=True)
    active = jnp.sum(ind, axis=1, keepdims=True)
    n_act = jnp.sum(active, axis=0, keepdims=True)
    jrow = jax.lax.broadcasted_iota(jnp.int32, (NT + 1, 1), 0)
    last = n_act - 1
    rowblk = jnp.where(active > 0, jrow, last)
    te_last = jnp.sum(te_row * (jrow == last).astype(jnp.int32), axis=0,
                      keepdims=True)
    te_c = jnp.where(active > 0, te_row, te_last)
    rowblk = jnp.where(jrow == NT, NT, rowblk)
    mode = jnp.where(jrow == NT, 2, active)
    meta = jnp.where(mlane == 0, te_c, 0)
    meta = jnp.where(mlane == 1, rowblk, meta)
    meta = jnp.where(mlane == 2, mode, meta)
    meta_ref[...] = meta


def _dispatch_body(x_hbm, scat_hbm, wrows_hbm, xg_hbm, ws_hbm,
                   idx_v, rows_v, wr_v, sem):
    wid = lax.axis_index("s") * NC + lax.axis_index("c")
    base = wid * CT
    pltpu.sync_copy(scat_hbm.at[wid], idx_v)
    stage = [pltpu.async_copy(x_hbm.at[pl.ds(base, CT)], rows_v, sem)]
    stage += [pltpu.async_copy(wrows_hbm.at[k, pl.ds(base, CT)], wr_v.at[k], sem)
              for k in range(K)]
    for c in stage:
        c.wait()
    copies = [pltpu.async_copy(rows_v, xg_hbm.at[idx_v.at[k]], sem)
              for k in range(K)]
    copies += [pltpu.async_copy(wr_v.at[k], ws_hbm.at[idx_v.at[k]], sem)
               for k in range(K)]
    for c in copies:
        c.wait()


def _sc_dispatch(x, scat3, wrows):
    mesh = plsc.VectorSubcoreMesh(core_axis_name="c", subcore_axis_name="s")
    return pl.kernel(
        _dispatch_body,
        out_type=[
            jax.ShapeDtypeStruct((S_MAX + BT + T, D), jnp.float32),
            jax.ShapeDtypeStruct((S_MAX + BT + T, 128), jnp.float32),
        ],
        mesh=mesh,
        scratch_types=[
            pltpu.VMEM((K, CT), jnp.int32),
            pltpu.VMEM((CT, D), jnp.float32),
            pltpu.VMEM((K, CT, 128), jnp.float32),
            pltpu.SemaphoreType.DMA,
        ],
    )(x, scat3, wrows)


def _ffn_body(te_ref, rb_ref, tv_ref, xg_ref, ws_ref, w1_ref, w3_ref, w2_ref,
              out_ref):
    j = pl.program_id(0)

    @pl.when(tv_ref[j] == 1)
    def _():
        xb = xg_ref[...].astype(jnp.bfloat16)
        h = jnp.dot(xb, w1_ref[0], preferred_element_type=jnp.float32)
        g = jnp.dot(xb, w3_ref[0], preferred_element_type=jnp.float32)
        act = (h * jax.nn.sigmoid(h)) * g
        out_ref[...] = jnp.dot(act.astype(jnp.bfloat16), w2_ref[0],
                               preferred_element_type=jnp.float32) * ws_ref[:, 0:1]

    @pl.when(tv_ref[j] == 2)
    def _():
        out_ref[...] = jnp.zeros((BT, D), jnp.float32)


def _grouped_ffn(xg, ws, w1b, w3b, w2b, te, rb, tv):
    grid_spec = pltpu.PrefetchScalarGridSpec(
        num_scalar_prefetch=3,
        grid=(NT + 1,),
        in_specs=[
            pl.BlockSpec((BT, D), lambda j, te, rb, tv: (rb[j], 0)),
            pl.BlockSpec((BT, 128), lambda j, te, rb, tv: (rb[j], 0)),
            pl.BlockSpec((1, D, F), lambda j, te, rb, tv: (te[j], 0, 0)),
            pl.BlockSpec((1, D, F), lambda j, te, rb, tv: (te[j], 0, 0)),
            pl.BlockSpec((1, F, D), lambda j, te, rb, tv: (te[j], 0, 0)),
        ],
        out_specs=pl.BlockSpec((BT, D), lambda j, te, rb, tv: (rb[j], 0)),
    )
    return pl.pallas_call(
        _ffn_body,
        grid_spec=grid_spec,
        out_shape=jax.ShapeDtypeStruct((S_MAX + BT, D), jnp.float32),
    )(te, rb, tv, xg, ws, w1b, w3b, w2b)


def _combine_body(ffn_hbm, gath_hbm, out_hbm, idx_v, rows_v, out_v, sem):
    wid = lax.axis_index("s") * NC + lax.axis_index("c")
    base = wid * CT
    pltpu.sync_copy(gath_hbm.at[wid], idx_v)

    def fire(c, buf):
        return [pltpu.async_copy(
            ffn_hbm.at[idx_v.at[k, pl.ds(c * CH, CH)]],
            rows_v.at[buf, k], sem) for k in range(K)]

    pending = fire(0, 0)
    for c in range(NCH):
        nxt = fire(c + 1, (c + 1) % 2) if c + 1 < NCH else []
        for cp in pending:
            cp.wait()
        pending = nxt
        buf = c % 2
        for i in range(CH):
            def gbody(gq, _):
                for u in range(4):
                    s = pl.ds((gq * 4 + u) * 16, 16)
                    out_v[i, s] = (rows_v[buf, 0, i, s] + rows_v[buf, 1, i, s]
                                   + rows_v[buf, 2, i, s])
                return 0
            lax.fori_loop(0, D // 64, gbody, 0)
        pltpu.sync_copy(out_v, out_hbm.at[pl.ds(base + c * CH, CH)])


def _sc_combine(ffn_out, gath3):
    mesh = plsc.VectorSubcoreMesh(core_axis_name="c", subcore_axis_name="s")
    return pl.kernel(
        _combine_body,
        out_type=jax.ShapeDtypeStruct((T, D), jnp.float32),
        mesh=mesh,
        scratch_types=[
            pltpu.VMEM((K, CT), jnp.int32),
            pltpu.VMEM((2, K, CH, D), jnp.float32),
            pltpu.VMEM((CH, D), jnp.float32),
            pltpu.SemaphoreType.DMA,
        ],
    )(ffn_out, gath3)


def kernel(x, Wg, W1, W3, W2):
    wg_pad = jnp.pad(Wg, ((0, 0), (0, 128 - E)))
    wrows, scat, gath, meta = pl.pallas_call(
        _route_body,
        out_shape=[
            jax.ShapeDtypeStruct((K, T, 128), jnp.float32),
            jax.ShapeDtypeStruct((T, 128), jnp.int32),
            jax.ShapeDtypeStruct((T, 128), jnp.int32),
            jax.ShapeDtypeStruct((NT + 1, 128), jnp.int32),
        ],
    )(x, wg_pad)

    scat3 = scat[:, :K].T.reshape(K, NW, CT).transpose(1, 0, 2)
    gath3 = gath[:, :K].T.reshape(K, NW, CT).transpose(1, 0, 2)
    te = meta[:, 0]
    rb = meta[:, 1]
    tv = meta[:, 2]

    xg, ws = _sc_dispatch(x, scat3, wrows)
    w1b = W1.astype(jnp.bfloat16)
    w3b = W3.astype(jnp.bfloat16)
    w2b = W2.astype(jnp.bfloat16)
    ffn_out = _grouped_ffn(xg, ws, w1b, w3b, w2b, te, rb, tv)
    return _sc_combine(ffn_out, gath3)


# speedup vs baseline: 1.3526x; 1.0742x over previous
import jax
import jax.numpy as jnp
from jax import lax
from jax.experimental import pallas as pl
from jax.experimental.pallas import tpu as pltpu
from jax.experimental.pallas import tpu_sc as plsc

E = 8
D = 1024
F = 2048
T = 2048
BT = 256
K = 3
NT = (K * T + E * (BT - 1) + BT - 1) // BT
S_MAX = NT * BT
NC = 2
NS = 16
NW = NC * NS
CT = T // NW
CH = 16
NCH = CT // CH


def _cumsum_lanes(a, width):
    sh = 1
    while sh < width:
        z = jnp.zeros(a.shape[:1] + (sh,), a.dtype)
        a = a + jnp.concatenate([z, a[:, :-sh]], axis=1)
        sh *= 2
    return a


def _cumsum_rows(a):
    n = a.shape[0]
    sh = 1
    while sh < n:
        z = jnp.zeros((sh,) + a.shape[1:], a.dtype)
        a = a + jnp.concatenate([z, a[:-sh]], axis=0)
        sh *= 2
    return a


def _route_body(x_ref, wg_ref, wrows_ref, scat_ref, gath_ref, meta_ref):
    logits = jnp.dot(x_ref[...], wg_ref[...], preferred_element_type=jnp.float32)
    lane = jax.lax.broadcasted_iota(jnp.int32, logits.shape, 1)
    valid = lane < E
    logits = jnp.where(valid, logits, jnp.float32(-1e30))
    lmax = jnp.max(logits, axis=1, keepdims=True)
    p = jnp.exp(logits - lmax)
    p = jnp.where(valid, p, 0.0)
    probs = p / jnp.sum(p, axis=1, keepdims=True)
    w0 = jnp.max(probs, axis=1, keepdims=True)
    cond0 = (probs == w0) & valid
    probs_m = jnp.where(cond0, 0.0, probs)
    w1v = jnp.max(probs_m, axis=1, keepdims=True)
    cond1 = (probs_m == w1v) & valid
    m = 1.0 / (w0 + w1v)
    weights = m * w0 * cond0.astype(jnp.float32) + m * w1v * cond1.astype(jnp.float32)

    sel = (cond0 | cond1).astype(jnp.int32)
    eexcl = _cumsum_lanes(sel, E) - sel
    sel3 = sel * (eexcl < K).astype(jnp.int32)
    tincl = _cumsum_rows(sel3)
    rk = tincl - sel3
    counts = tincl[-1:, :]
    pc = (counts + (BT - 1)) // BT * BT
    goff = _cumsum_lanes(pc, E) - pc
    slot = goff + rk

    tok = jax.lax.broadcasted_iota(jnp.int32, (T, 1), 0)
    scat = jnp.zeros((T, 128), jnp.int32)
    gath = jnp.zeros((T, 128), jnp.int32)
    for k in range(K):
        hot = (sel3 * (eexcl == k).astype(jnp.int32)).astype(jnp.float32)
        live = jnp.sum(hot, axis=1, keepdims=True) > 0
        slot_k = jnp.sum(hot * slot.astype(jnp.float32), axis=1,
                         keepdims=True).astype(jnp.int32)
        w_k = jnp.sum(hot * weights, axis=1, keepdims=True)
        w_k = jnp.where(live, w_k, 0.0)
        scat_k = jnp.where(live, slot_k, S_MAX + BT + tok)
        gath_k = jnp.where(live, slot_k, S_MAX + (tok % BT))
        wrows_ref[k] = jnp.broadcast_to(w_k, (T, 128))
        is_k = lane == k
        scat = jnp.where(is_k, scat_k, scat)
        gath = jnp.where(is_k, gath_k, gath)
    scat_ref[...] = scat
    gath_ref[...] = gath

    jv = jax.lax.broadcasted_iota(jnp.int32, (NT + 1, 128), 0) * BT
    mlane = jax.lax.broadcasted_iota(jnp.int32, (NT + 1, 128), 1)
    ind = ((goff <= jv) & (jv < goff + pc) & (mlane < E)).astype(jnp.int32)
    te_row = jnp.sum(ind * mlane, axis=1, keepdims=True)
    active = jnp.sum(ind, axis=1, keepdims=True)
    n_act = jnp.sum(active, axis=0, keepdims=True)
    jrow = jax.lax.broadcasted_iota(jnp.int32, (NT + 1, 1), 0)
    last = n_act - 1
    rowblk = jnp.where(active > 0, jrow, last)
    te_last = jnp.sum(te_row * (jrow == last).astype(jnp.int32), axis=0,
                      keepdims=True)
    te_c = jnp.where(active > 0, te_row, te_last)
    rowblk = jnp.where(jrow == NT, NT, rowblk)
    mode = jnp.where(jrow == NT, 2, active)
    meta = jnp.where(mlane == 0, te_c, 0)
    meta = jnp.where(mlane == 1, rowblk, meta)
    meta = jnp.where(mlane == 2, mode, meta)
    meta_ref[...] = meta


def _dispatch_body(x_hbm, scat_hbm, wrows_hbm, xg_hbm, ws_hbm,
                   idx_v, rows_v, wr_v, sem):
    wid = lax.axis_index("s") * NC + lax.axis_index("c")
    base = wid * CT
    pltpu.sync_copy(scat_hbm.at[wid], idx_v)
    stage = [pltpu.async_copy(x_hbm.at[pl.ds(base, CT)], rows_v, sem)]
    stage += [pltpu.async_copy(wrows_hbm.at[k, pl.ds(base, CT)], wr_v.at[k], sem)
              for k in range(K)]
    for c in stage:
        c.wait()
    copies = [pltpu.async_copy(rows_v, xg_hbm.at[idx_v.at[k]], sem)
              for k in range(K)]
    copies += [pltpu.async_copy(wr_v.at[k], ws_hbm.at[idx_v.at[k]], sem)
               for k in range(K)]
    for c in copies:
        c.wait()


def _sc_dispatch(x, scat3, wrows):
    mesh = plsc.VectorSubcoreMesh(core_axis_name="c", subcore_axis_name="s")
    return pl.kernel(
        _dispatch_body,
        out_type=[
            jax.ShapeDtypeStruct((S_MAX + BT + T, D), jnp.float32),
            jax.ShapeDtypeStruct((S_MAX + BT + T, 128), jnp.float32),
        ],
        mesh=mesh,
        scratch_types=[
            pltpu.VMEM((K, CT), jnp.int32),
            pltpu.VMEM((CT, D), jnp.float32),
            pltpu.VMEM((K, CT, 128), jnp.float32),
            pltpu.SemaphoreType.DMA,
        ],
    )(x, scat3, wrows)


def _ffn_body(te_ref, rb_ref, tv_ref, xg_ref, ws_ref, w1_ref, w3_ref, w2_ref,
              out_ref):
    j = pl.program_id(0)

    @pl.when(tv_ref[j] == 1)
    def _():
        xb = xg_ref[...].astype(jnp.bfloat16)
        h = jnp.dot(xb, w1_ref[0], preferred_element_type=jnp.float32)
        g = jnp.dot(xb, w3_ref[0].astype(jnp.bfloat16),
                    preferred_element_type=jnp.float32)
        act = (h * jax.nn.sigmoid(h)) * g
        out_ref[...] = jnp.dot(act.astype(jnp.bfloat16), w2_ref[0],
                               preferred_element_type=jnp.float32) * ws_ref[:, 0:1]

    @pl.when(tv_ref[j] == 2)
    def _():
        out_ref[...] = jnp.zeros((BT, D), jnp.float32)


def _grouped_ffn(xg, ws, w1b, w3b, w2b, te, rb, tv):
    grid_spec = pltpu.PrefetchScalarGridSpec(
        num_scalar_prefetch=3,
        grid=(NT + 1,),
        in_specs=[
            pl.BlockSpec((BT, D), lambda j, te, rb, tv: (rb[j], 0)),
            pl.BlockSpec((BT, 128), lambda j, te, rb, tv: (rb[j], 0)),
            pl.BlockSpec((1, D, F), lambda j, te, rb, tv: (te[j], 0, 0)),
            pl.BlockSpec((1, D, F), lambda j, te, rb, tv: (te[j], 0, 0)),
            pl.BlockSpec((1, F, D), lambda j, te, rb, tv: (te[j], 0, 0)),
        ],
        out_specs=pl.BlockSpec((BT, D), lambda j, te, rb, tv: (rb[j], 0)),
    )
    return pl.pallas_call(
        _ffn_body,
        grid_spec=grid_spec,
        out_shape=jax.ShapeDtypeStruct((S_MAX + BT, D), jnp.float32),
    )(te, rb, tv, xg, ws, w1b, w3b, w2b)


def _combine_body(ffn_hbm, gath_hbm, out_hbm, idx_v, rows_v, out_v, sem):
    wid = lax.axis_index("s") * NC + lax.axis_index("c")
    base = wid * CT
    pltpu.sync_copy(gath_hbm.at[wid], idx_v)

    def fire(c, buf):
        return [pltpu.async_copy(
            ffn_hbm.at[idx_v.at[k, pl.ds(c * CH, CH)]],
            rows_v.at[buf, k], sem) for k in range(K)]

    pending = fire(0, 0)
    for c in range(NCH):
        nxt = fire(c + 1, (c + 1) % 2) if c + 1 < NCH else []
        for cp in pending:
            cp.wait()
        pending = nxt
        buf = c % 2
        for i in range(CH):
            def gbody(gq, _):
                for u in range(4):
                    s = pl.ds((gq * 4 + u) * 16, 16)
                    out_v[i, s] = (rows_v[buf, 0, i, s] + rows_v[buf, 1, i, s]
                                   + rows_v[buf, 2, i, s])
                return 0
            lax.fori_loop(0, D // 64, gbody, 0)
        pltpu.sync_copy(out_v, out_hbm.at[pl.ds(base + c * CH, CH)])


def _sc_combine(ffn_out, gath3):
    mesh = plsc.VectorSubcoreMesh(core_axis_name="c", subcore_axis_name="s")
    return pl.kernel(
        _combine_body,
        out_type=jax.ShapeDtypeStruct((T, D), jnp.float32),
        mesh=mesh,
        scratch_types=[
            pltpu.VMEM((K, CT), jnp.int32),
            pltpu.VMEM((2, K, CH, D), jnp.float32),
            pltpu.VMEM((CH, D), jnp.float32),
            pltpu.SemaphoreType.DMA,
        ],
    )(ffn_out, gath3)


def kernel(x, Wg, W1, W3, W2):
    wg_pad = jnp.pad(Wg, ((0, 0), (0, 128 - E)))
    wrows, scat, gath, meta = pl.pallas_call(
        _route_body,
        out_shape=[
            jax.ShapeDtypeStruct((K, T, 128), jnp.float32),
            jax.ShapeDtypeStruct((T, 128), jnp.int32),
            jax.ShapeDtypeStruct((T, 128), jnp.int32),
            jax.ShapeDtypeStruct((NT + 1, 128), jnp.int32),
        ],
    )(x, wg_pad)

    scat3 = scat[:, :K].T.reshape(K, NW, CT).transpose(1, 0, 2)
    gath3 = gath[:, :K].T.reshape(K, NW, CT).transpose(1, 0, 2)
    te = meta[:, 0]
    rb = meta[:, 1]
    tv = meta[:, 2]

    xg, ws = _sc_dispatch(x, scat3, wrows)
    w1b = W1.astype(jnp.bfloat16)
    w2b = W2.astype(jnp.bfloat16)
    ffn_out = _grouped_ffn(xg, ws, w1b, W3, w2b, te, rb, tv)
    return _sc_combine(ffn_out, gath3)
